# baseline (speedup 1.0000x reference)
import functools

import jax
import jax.numpy as jnp
from jax.experimental import pallas as pl
from jax.experimental.pallas import tpu as pltpu

_MAX_LEN = 2048
_NSEG = 4


def _seg_bounds(seq_len):
    q = seq_len // _NSEG
    bounds = [0, q + 8]
    for _ in range(_NSEG - 2):
        bounds.append(bounds[-1] + q)
    bounds.append(seq_len)
    return bounds


def _body(x_ref, emb_hbm, o_ref, emb_vmem, tail_vmem, *sems, seq_len, base, shift):
    b = pl.program_id(1)
    q = seq_len // _NSEG
    bounds = _seg_bounds(seq_len)
    sem_t = sems[_NSEG]

    def _seg_copy(k):
        lo, hi = bounds[k], bounds[k + 1]
        return pltpu.make_async_copy(
            emb_hbm.at[pl.ds(base + lo, hi - lo), :],
            emb_vmem.at[pl.ds(lo, hi - lo), :],
            sems[k],
        )

    def _tail_copy():
        return pltpu.make_async_copy(
            emb_hbm.at[pl.ds(base + seq_len, shift), :], tail_vmem, sem_t
        )

    @pl.when(b == 0)
    def _start_dmas():
        for k in range(_NSEG):
            _seg_copy(k).start()
        _tail_copy().start()

    for k in range(_NSEG):
        @pl.when(b == 0)
        def _wait_seg(k=k):
            _seg_copy(k).wait()
            if k == _NSEG - 1:
                _tail_copy().wait()
                pad = jnp.zeros((8 - shift, tail_vmem.shape[1]), tail_vmem.dtype)
                emb_vmem[pl.ds(seq_len, 8), :] = jnp.concatenate(
                    [tail_vmem[...], pad], axis=0
                )

        h0 = k * q
        win = emb_vmem[pl.ds(h0, q + 8), :]
        rows = jax.lax.slice(win, (shift, 0), (shift + q, win.shape[1]))
        o_ref[0, pl.ds(h0, q), :] = x_ref[0, pl.ds(h0, q), :] + rows


def kernel(x, rel_pos_emb):
    batch, seq_len, d_model = x.shape
    base = (_MAX_LEN - 1) // 8 * 8
    shift = (_MAX_LEN - 1) - base
    body = functools.partial(_body, seq_len=seq_len, base=base, shift=shift)
    return pl.pallas_call(
        body,
        grid=(1, batch),
        in_specs=[
            pl.BlockSpec((1, seq_len, d_model), lambda i, b: (b, i, 0)),
            pl.BlockSpec(memory_space=pltpu.MemorySpace.HBM),
        ],
        out_specs=pl.BlockSpec((1, seq_len, d_model), lambda i, b: (b, i, 0)),
        out_shape=jax.ShapeDtypeStruct(x.shape, x.dtype),
        scratch_shapes=[
            pltpu.VMEM((seq_len + 8, d_model), x.dtype),
            pltpu.VMEM((shift, d_model), x.dtype),
        ] + [pltpu.SemaphoreType.DMA] * (_NSEG + 1),
    )(x, rel_pos_emb)
